# baseline (speedup 1.0000x reference)
import functools

import jax
import jax.numpy as jnp
from jax import lax
from jax.experimental import pallas as pl
from jax.experimental.pallas import tpu as pltpu
from jax.experimental.pallas import tpu_sc as plsc

L = 16
CH = 64


def _sc_broadcast(h, w, d, f, bs):
    n_rows = bs * 3 * h * w * d
    rows_blk = w * d
    mesh = plsc.VectorSubcoreMesh(
        core_axis_name="c", subcore_axis_name="s", num_cores=2
    )

    @functools.partial(
        pl.kernel,
        mesh=mesh,
        out_type=jax.ShapeDtypeStruct((n_rows, f), jnp.float32),
        scratch_types=[
            pltpu.VMEM((h, f), jnp.float32),
            pltpu.VMEM((w, f), jnp.float32),
            pltpu.VMEM((d, f), jnp.float32),
            pltpu.VMEM((CH, f), jnp.float32),
            pltpu.VMEM((2, CH, f), jnp.float32),
            pltpu.SemaphoreType.DMA,
            pltpu.SemaphoreType.DMA,
            pltpu.SemaphoreType.DMA,
            pltpu.SemaphoreType.DMA,
        ],
    )
    def run(xt_hbm, yt_hbm, zt_hbm, out_hbm, xt_v, yt_v, zt_v,
            pbx, pby, semx, semy0, semy1, semz):
        wid = lax.axis_index("s") * 2 + lax.axis_index("c")
        pltpu.sync_copy(xt_hbm, xt_v)
        pltpu.sync_copy(yt_hbm, yt_v)
        pltpu.sync_copy(zt_hbm, zt_v)
        semy = [semy0, semy1]

        def z_body(m, _):
            b = lax.shift_right_logical(m, 5)
            ch = lax.bitwise_and(m, 31)
            g = (b * 3 + 2) * h + wid
            row0 = g * rows_blk + ch * d
            pltpu.async_copy(zt_v, out_hbm.at[pl.ds(row0, d)], semz)
            return 0
        lax.fori_loop(0, 2 * (rows_blk // d), z_body, 0)

        xv = [xt_v[wid, pl.ds(q * L, L)] for q in range(f // L)]

        def fx_body(r, _):
            for q in range(f // L):
                pbx[r, pl.ds(q * L, L)] = xv[q]
            return 0
        lax.fori_loop(0, CH, fx_body, 0)

        def x_body(m, _):
            b = lax.shift_right_logical(m, 4)
            ch = lax.bitwise_and(m, 15)
            g = b * 3 * h + wid
            row0 = g * rows_blk + ch * CH
            pltpu.async_copy(pbx, out_hbm.at[pl.ds(row0, CH)], semx)
            return 0
        lax.fori_loop(0, 2 * (rows_blk // CH), x_body, 0)

        n_ch = rows_blk // CH
        jpc = CH // d

        def y_pair(t, _):
            for par in (0, 1):
                c = 2 * t + par

                @pl.when(t >= 1)
                def _():
                    pltpu.make_async_copy(
                        pby.at[par], out_hbm.at[pl.ds(0, CH)], semy[par]
                    ).wait()
                    pltpu.make_async_copy(
                        pby.at[par], out_hbm.at[pl.ds(0, CH)], semy[par]
                    ).wait()

                def fy_body(r, _, c=c, par=par):
                    j = c * jpc + lax.shift_right_logical(r, 5)
                    for q in range(f // L):
                        pby[par, r, pl.ds(q * L, L)] = yt_v[j, pl.ds(q * L, L)]
                    return 0
                lax.fori_loop(0, CH, fy_body, 0)

                for b in range(bs):
                    g = (b * 3 + 1) * h + wid
                    row0 = g * rows_blk + c * CH
                    pltpu.async_copy(
                        pby.at[par], out_hbm.at[pl.ds(row0, CH)], semy[par]
                    )
            return 0
        lax.fori_loop(0, n_ch // 2, y_pair, 0)

        for _ in range(2):
            for par in (0, 1):
                pltpu.make_async_copy(
                    pby.at[par], out_hbm.at[pl.ds(0, CH)], semy[par]
                ).wait()

        def drain_x(m, _):
            pltpu.make_async_copy(
                pbx, out_hbm.at[pl.ds(0, CH)], semx
            ).wait()
            return 0
        lax.fori_loop(0, 2 * (rows_blk // CH), drain_x, 0)

        def drain_z(m, _):
            pltpu.make_async_copy(
                zt_v, out_hbm.at[pl.ds(0, d)], semz
            ).wait()
            return 0
        lax.fori_loop(0, 2 * (rows_blk // d), drain_z, 0)

    return run


@jax.jit
def kernel(x, x_table, y_table, z_table):
    bs, _, h, w, d = x.shape
    f = x_table.shape[-1]
    flat = _sc_broadcast(h, w, d, f, bs)(x_table, y_table, z_table)
    return flat.reshape(bs, 3, h, w, d, f)


# speedup vs baseline: 1.1233x; 1.1233x over previous
import functools

import jax
import jax.numpy as jnp
from jax import lax
from jax.experimental import pallas as pl
from jax.experimental.pallas import tpu as pltpu
from jax.experimental.pallas import tpu_sc as plsc

L = 16
CH = 128


def _sc_broadcast(h, w, d, f, bs):
    n_rows = bs * 3 * h * w * d
    rows_blk = w * d
    mesh = plsc.VectorSubcoreMesh(
        core_axis_name="c", subcore_axis_name="s", num_cores=2
    )

    @functools.partial(
        pl.kernel,
        mesh=mesh,
        out_type=jax.ShapeDtypeStruct((n_rows, f), jnp.float32),
        scratch_types=[
            pltpu.VMEM((h, f), jnp.float32),
            pltpu.VMEM((w, f), jnp.float32),
            pltpu.VMEM((d, f), jnp.float32),
            pltpu.VMEM((CH, f), jnp.float32),
            pltpu.VMEM((2, CH, f), jnp.float32),
            pltpu.SemaphoreType.DMA,
            pltpu.SemaphoreType.DMA,
            pltpu.SemaphoreType.DMA,
            pltpu.SemaphoreType.DMA,
        ],
    )
    def run(xt_hbm, yt_hbm, zt_hbm, out_hbm, xt_v, yt_v, zt_v,
            pbx, pby, semx, semy0, semy1, semz):
        wid = lax.axis_index("s") * 2 + lax.axis_index("c")
        pltpu.sync_copy(xt_hbm, xt_v)
        pltpu.sync_copy(yt_hbm, yt_v)
        pltpu.sync_copy(zt_hbm, zt_v)
        semy = [semy0, semy1]

        def z_body(m, _):
            b = lax.shift_right_logical(m, 5)
            ch = lax.bitwise_and(m, 31)
            g = (b * 3 + 2) * h + wid
            row0 = g * rows_blk + ch * d
            pltpu.async_copy(zt_v, out_hbm.at[pl.ds(row0, d)], semz)
            return 0
        lax.fori_loop(0, 2 * (rows_blk // d), z_body, 0)

        xv = [xt_v[wid, pl.ds(q * L, L)] for q in range(f // L)]

        def fx_body(r, _):
            for q in range(f // L):
                pbx[r, pl.ds(q * L, L)] = xv[q]
            return 0
        lax.fori_loop(0, CH, fx_body, 0)

        n_chx = rows_blk // CH

        def x_body(m, _):
            b = m // n_chx
            ch = lax.rem(m, n_chx)
            g = b * 3 * h + wid
            row0 = g * rows_blk + ch * CH
            pltpu.async_copy(pbx, out_hbm.at[pl.ds(row0, CH)], semx)
            return 0
        lax.fori_loop(0, 2 * (rows_blk // CH), x_body, 0)

        n_ch = rows_blk // CH
        jpc = CH // d
        for c in range(n_ch):
            par = c % 2
            if c >= 2:
                for _ in range(2):
                    pltpu.make_async_copy(
                        pby.at[par], out_hbm.at[pl.ds(0, CH)], semy[par]
                    ).wait()

            def fy_body(r, _, c=c, par=par):
                j = c * jpc + lax.shift_right_logical(r, 5)
                for q in range(f // L):
                    pby[par, r, pl.ds(q * L, L)] = yt_v[j, pl.ds(q * L, L)]
                return 0
            lax.fori_loop(0, CH, fy_body, 0)

            for b in range(bs):
                g = (b * 3 + 1) * h + wid
                row0 = g * rows_blk + c * CH
                pltpu.async_copy(
                    pby.at[par], out_hbm.at[pl.ds(row0, CH)], semy[par]
                )

        for _ in range(2):
            for par in (0, 1):
                pltpu.make_async_copy(
                    pby.at[par], out_hbm.at[pl.ds(0, CH)], semy[par]
                ).wait()

        def drain_x(m, _):
            pltpu.make_async_copy(
                pbx, out_hbm.at[pl.ds(0, CH)], semx
            ).wait()
            return 0
        lax.fori_loop(0, 2 * (rows_blk // CH), drain_x, 0)

        def drain_z(m, _):
            pltpu.make_async_copy(
                zt_v, out_hbm.at[pl.ds(0, d)], semz
            ).wait()
            return 0
        lax.fori_loop(0, 2 * (rows_blk // d), drain_z, 0)

    return run


@jax.jit
def kernel(x, x_table, y_table, z_table):
    bs, _, h, w, d = x.shape
    f = x_table.shape[-1]
    flat = _sc_broadcast(h, w, d, f, bs)(x_table, y_table, z_table)
    return flat.reshape(bs, 3, h, w, d, f)
